# baseline (speedup 1.0000x reference)
import jax
import jax.numpy as jnp
from jax.experimental import pallas as pl
from jax.experimental.pallas import tpu as pltpu


def _scse_plane_kernel(x_ref, w1s_ref, b1_ref, w2_ref, b2_ref, ws_ref, bs_ref,
                       o_ref):
    x = x_ref[0]

    csum = jnp.sum(x, axis=1, keepdims=True)
    z = jnp.sum(csum * w1s_ref[...], axis=0, keepdims=True)
    z = jnp.maximum(z + b1_ref[...], 0.0)
    s = jnp.sum(w2_ref[...] * z, axis=1, keepdims=True)
    s = jax.nn.sigmoid(s + b2_ref[...])

    q = jnp.sum(x * ws_ref[...], axis=0, keepdims=True)
    q = jax.nn.sigmoid(q + bs_ref[0])

    o_ref[0] = x * (s + q)


def kernel(x_nchw, w1, b1, w2, b2, ws, bs):
    N, C, H, W = x_nchw.shape
    HW = H * W
    dtype = x_nchw.dtype
    x = x_nchw.reshape(N, C, HW)

    HWp = ((HW + 127) // 128) * 128
    if HWp != HW:
        x = jnp.pad(x, ((0, 0), (0, 0), (0, HWp - HW)))

    w1s = w1 * (1.0 / float(HW))

    out = pl.pallas_call(
        _scse_plane_kernel,
        out_shape=jax.ShapeDtypeStruct((N, C, HWp), dtype),
        grid=(N,),
        in_specs=[
            pl.BlockSpec((1, C, HWp), lambda n: (n, 0, 0)),
            pl.BlockSpec(w1s.shape, lambda n: (0, 0)),
            pl.BlockSpec(b1.shape, lambda n: (0, 0)),
            pl.BlockSpec(w2.shape, lambda n: (0, 0)),
            pl.BlockSpec(b2.shape, lambda n: (0, 0)),
            pl.BlockSpec(ws.shape, lambda n: (0, 0)),
            pl.BlockSpec(memory_space=pltpu.MemorySpace.SMEM),
        ],
        out_specs=pl.BlockSpec((1, C, HWp), lambda n: (n, 0, 0)),
        compiler_params=pltpu.CompilerParams(
            dimension_semantics=("parallel",),
            vmem_limit_bytes=40 * 1024 * 1024,
        ),
        cost_estimate=pl.CostEstimate(
            flops=6 * N * C * HWp,
            transcendentals=N * (HWp + C),
            bytes_accessed=2 * N * C * HWp * dtype.itemsize,
        ),
    )(x, w1s, b1, w2, b2, ws, bs)

    if HWp != HW:
        out = out[:, :, :HW]
    return out.reshape(N, C, H, W)


# speedup vs baseline: 1.0062x; 1.0062x over previous
import jax
import jax.numpy as jnp
from jax.experimental import pallas as pl
from jax.experimental.pallas import tpu as pltpu


def _scse_block_kernel(x_ref, w1s_ref, b1_ref, w2_ref, b2_ref, ws_ref, bs_ref,
                       o_ref, *, nb):
    for i in range(nb):
        x = x_ref[i]

        csum = jnp.sum(x, axis=1, keepdims=True)
        z = jnp.sum(csum * w1s_ref[...], axis=0, keepdims=True)
        z = jnp.maximum(z + b1_ref[...], 0.0)
        s = jnp.sum(w2_ref[...] * z, axis=1, keepdims=True)
        s = jax.nn.sigmoid(s + b2_ref[...])

        q = jnp.sum(x * ws_ref[...], axis=0, keepdims=True)
        q = jax.nn.sigmoid(q + bs_ref[0])

        o_ref[i] = x * (s + q)


def kernel(x_nchw, w1, b1, w2, b2, ws, bs):
    import functools

    N, C, H, W = x_nchw.shape
    HW = H * W
    dtype = x_nchw.dtype
    x = x_nchw.reshape(N, C, HW)

    HWp = ((HW + 127) // 128) * 128
    if HWp != HW:
        x = jnp.pad(x, ((0, 0), (0, 0), (0, HWp - HW)))

    w1s = w1 * (1.0 / float(HW))

    plane_bytes = C * HWp * dtype.itemsize
    nb = 1
    for cand in (4, 2, 1):
        if N % cand == 0 and 4 * cand * plane_bytes <= 44 * 1024 * 1024:
            nb = cand
            break

    body = functools.partial(_scse_block_kernel, nb=nb)
    out = pl.pallas_call(
        body,
        out_shape=jax.ShapeDtypeStruct((N, C, HWp), dtype),
        grid=(N // nb,),
        in_specs=[
            pl.BlockSpec((nb, C, HWp), lambda n: (n, 0, 0)),
            pl.BlockSpec(w1s.shape, lambda n: (0, 0)),
            pl.BlockSpec(b1.shape, lambda n: (0, 0)),
            pl.BlockSpec(w2.shape, lambda n: (0, 0)),
            pl.BlockSpec(b2.shape, lambda n: (0, 0)),
            pl.BlockSpec(ws.shape, lambda n: (0, 0)),
            pl.BlockSpec(memory_space=pltpu.MemorySpace.SMEM),
        ],
        out_specs=pl.BlockSpec((nb, C, HWp), lambda n: (n, 0, 0)),
        compiler_params=pltpu.CompilerParams(
            dimension_semantics=("parallel",),
            vmem_limit_bytes=52 * 1024 * 1024,
        ),
        cost_estimate=pl.CostEstimate(
            flops=6 * N * C * HWp,
            transcendentals=N * (HWp + C),
            bytes_accessed=2 * N * C * HWp * dtype.itemsize,
        ),
    )(x, w1s, b1, w2, b2, ws, bs)

    if HWp != HW:
        out = out[:, :, :HW]
    return out.reshape(N, C, H, W)


# speedup vs baseline: 1.0077x; 1.0015x over previous
import jax
import jax.numpy as jnp
from jax.experimental import pallas as pl
from jax.experimental.pallas import tpu as pltpu


def _scse_block_kernel(x_ref, w1_ref, b1_ref, w2_ref, b2_ref, ws_ref, bs_ref,
                       o_ref, *, nb, inv_hw):
    for i in range(nb):
        x = x_ref[i]

        mean = jnp.sum(x, axis=1, keepdims=True) * inv_hw
        z = jnp.sum(mean * w1_ref[...], axis=0, keepdims=True)
        z = jnp.maximum(z + b1_ref[...], 0.0)
        s = jnp.sum(w2_ref[...] * z, axis=1, keepdims=True)
        s = jax.nn.sigmoid(s + b2_ref[...])

        q = jnp.sum(x * ws_ref[...], axis=0, keepdims=True)
        q = jax.nn.sigmoid(q + bs_ref[0])

        o_ref[i] = x * (s + q)


def kernel(x_nchw, w1, b1, w2, b2, ws, bs):
    import functools

    N, C, H, W = x_nchw.shape
    HW = H * W
    dtype = x_nchw.dtype
    x = x_nchw.reshape(N, C, HW)

    HWp = ((HW + 127) // 128) * 128
    if HWp != HW:
        x = jnp.pad(x, ((0, 0), (0, 0), (0, HWp - HW)))

    plane_bytes = C * HWp * dtype.itemsize
    nb = 1
    for cand in (4, 2, 1):
        if N % cand == 0 and 4 * cand * plane_bytes <= 44 * 1024 * 1024:
            nb = cand
            break

    body = functools.partial(_scse_block_kernel, nb=nb, inv_hw=1.0 / float(HW))
    out = pl.pallas_call(
        body,
        out_shape=jax.ShapeDtypeStruct((N, C, HWp), dtype),
        grid=(N // nb,),
        in_specs=[
            pl.BlockSpec((nb, C, HWp), lambda n: (n, 0, 0)),
            pl.BlockSpec(w1.shape, lambda n: (0, 0)),
            pl.BlockSpec(b1.shape, lambda n: (0, 0)),
            pl.BlockSpec(w2.shape, lambda n: (0, 0)),
            pl.BlockSpec(b2.shape, lambda n: (0, 0)),
            pl.BlockSpec(ws.shape, lambda n: (0, 0)),
            pl.BlockSpec(memory_space=pltpu.MemorySpace.SMEM),
        ],
        out_specs=pl.BlockSpec((nb, C, HWp), lambda n: (n, 0, 0)),
        compiler_params=pltpu.CompilerParams(
            dimension_semantics=("parallel",),
            vmem_limit_bytes=52 * 1024 * 1024,
        ),
        cost_estimate=pl.CostEstimate(
            flops=6 * N * C * HWp,
            transcendentals=N * (HWp + C),
            bytes_accessed=2 * N * C * HWp * dtype.itemsize,
        ),
    )(x, w1, b1, w2, b2, ws, bs)

    if HWp != HW:
        out = out[:, :, :HW]
    return out.reshape(N, C, H, W)


# speedup vs baseline: 1.0087x; 1.0010x over previous
import functools

import jax
import jax.numpy as jnp
from jax.experimental import pallas as pl
from jax.experimental.pallas import tpu as pltpu


def _scse_block_kernel(x_ref, w1_ref, b1_ref, w2_ref, b2_ref, ws_ref, bs_ref,
                       o_ref, *, nb, hw, tw, inv_hw):
    ws = ws_ref[...]
    for i in range(nb):
        x = x_ref[i]

        mean = jnp.sum(x, axis=1, keepdims=True) * inv_hw
        z = jnp.sum(mean * w1_ref[...], axis=0, keepdims=True)
        z = jnp.maximum(z + b1_ref[...], 0.0)
        s = jnp.sum(w2_ref[...] * z, axis=1, keepdims=True)
        s = jax.nn.sigmoid(s + b2_ref[...])

        for t in range(0, hw, tw):
            xt = x[:, t:t + tw]
            qt = jnp.sum(xt * ws, axis=0, keepdims=True)
            qt = jax.nn.sigmoid(qt + bs_ref[0])
            o_ref[i, :, t:t + tw] = xt * (s + qt)


def kernel(x_nchw, w1, b1, w2, b2, ws, bs):
    N, C, H, W = x_nchw.shape
    HW = H * W
    dtype = x_nchw.dtype
    x = x_nchw.reshape(N, C, HW)

    HWp = ((HW + 127) // 128) * 128
    if HWp != HW:
        x = jnp.pad(x, ((0, 0), (0, 0), (0, HWp - HW)))

    plane_bytes = C * HWp * dtype.itemsize
    nb = 1
    for cand in (4, 2, 1):
        if N % cand == 0 and 4 * cand * plane_bytes <= 44 * 1024 * 1024:
            nb = cand
            break

    tw = 128

    body = functools.partial(_scse_block_kernel, nb=nb, hw=HWp, tw=tw,
                             inv_hw=1.0 / float(HW))
    out = pl.pallas_call(
        body,
        out_shape=jax.ShapeDtypeStruct((N, C, HWp), dtype),
        grid=(N // nb,),
        in_specs=[
            pl.BlockSpec((nb, C, HWp), lambda n: (n, 0, 0)),
            pl.BlockSpec(w1.shape, lambda n: (0, 0)),
            pl.BlockSpec(b1.shape, lambda n: (0, 0)),
            pl.BlockSpec(w2.shape, lambda n: (0, 0)),
            pl.BlockSpec(b2.shape, lambda n: (0, 0)),
            pl.BlockSpec(ws.shape, lambda n: (0, 0)),
            pl.BlockSpec(memory_space=pltpu.MemorySpace.SMEM),
        ],
        out_specs=pl.BlockSpec((nb, C, HWp), lambda n: (n, 0, 0)),
        compiler_params=pltpu.CompilerParams(
            dimension_semantics=("parallel",),
            vmem_limit_bytes=52 * 1024 * 1024,
        ),
        cost_estimate=pl.CostEstimate(
            flops=6 * N * C * HWp,
            transcendentals=N * (HWp + C),
            bytes_accessed=2 * N * C * HWp * dtype.itemsize,
        ),
    )(x, w1, b1, w2, b2, ws, bs)

    if HWp != HW:
        out = out[:, :, :HW]
    return out.reshape(N, C, H, W)


# speedup vs baseline: 1.0089x; 1.0001x over previous
import functools

import jax
import jax.numpy as jnp
from jax.experimental import pallas as pl
from jax.experimental.pallas import tpu as pltpu


def _scse_block_kernel(x_ref, w1_ref, b1_ref, w2_ref, b2_ref, ws_ref, bs_ref,
                       o_ref, *, nb, hw, tw, inv_hw):
    ws = ws_ref[...]
    for i in range(nb):
        x = x_ref[i]

        mean = jnp.sum(x, axis=1, keepdims=True) * inv_hw
        z = jnp.sum(mean * w1_ref[...], axis=0, keepdims=True)
        z = jnp.maximum(z + b1_ref[...], 0.0)
        s = jnp.sum(w2_ref[...] * z, axis=1, keepdims=True)
        s = jax.nn.sigmoid(s + b2_ref[...])

        q = jax.lax.dot_general(ws, x, (((0,), (0,)), ((), ())),
                                preferred_element_type=jnp.float32)
        q = jax.nn.sigmoid(q + bs_ref[0])

        for t in range(0, hw, tw):
            o_ref[i, :, t:t + tw] = x[:, t:t + tw] * (s + q[:, t:t + tw])


def kernel(x_nchw, w1, b1, w2, b2, ws, bs):
    N, C, H, W = x_nchw.shape
    HW = H * W
    dtype = x_nchw.dtype
    x = x_nchw.reshape(N, C, HW)

    HWp = ((HW + 127) // 128) * 128
    if HWp != HW:
        x = jnp.pad(x, ((0, 0), (0, 0), (0, HWp - HW)))

    plane_bytes = C * HWp * dtype.itemsize
    nb = 1
    for cand in (4, 2, 1):
        if N % cand == 0 and 4 * cand * plane_bytes <= 44 * 1024 * 1024:
            nb = cand
            break

    tw = 128

    body = functools.partial(_scse_block_kernel, nb=nb, hw=HWp, tw=tw,
                             inv_hw=1.0 / float(HW))
    out = pl.pallas_call(
        body,
        out_shape=jax.ShapeDtypeStruct((N, C, HWp), dtype),
        grid=(N // nb,),
        in_specs=[
            pl.BlockSpec((nb, C, HWp), lambda n: (n, 0, 0)),
            pl.BlockSpec(w1.shape, lambda n: (0, 0)),
            pl.BlockSpec(b1.shape, lambda n: (0, 0)),
            pl.BlockSpec(w2.shape, lambda n: (0, 0)),
            pl.BlockSpec(b2.shape, lambda n: (0, 0)),
            pl.BlockSpec(ws.shape, lambda n: (0, 0)),
            pl.BlockSpec(memory_space=pltpu.MemorySpace.SMEM),
        ],
        out_specs=pl.BlockSpec((nb, C, HWp), lambda n: (n, 0, 0)),
        compiler_params=pltpu.CompilerParams(
            dimension_semantics=("parallel",),
            vmem_limit_bytes=52 * 1024 * 1024,
        ),
        cost_estimate=pl.CostEstimate(
            flops=6 * N * C * HWp,
            transcendentals=N * (HWp + C),
            bytes_accessed=2 * N * C * HWp * dtype.itemsize,
        ),
    )(x, w1, b1, w2, b2, ws, bs)

    if HWp != HW:
        out = out[:, :, :HW]
    return out.reshape(N, C, H, W)
